# baseline (speedup 1.0000x reference)
import jax
import jax.numpy as jnp
import numpy as np
from jax import lax
from jax.experimental import pallas as pl
from jax.experimental.pallas import tpu as pltpu
from jax.experimental.pallas import tpu_sc as plsc

_NC = 2
_NS = 16
_NW = _NC * _NS
_L = 16
_CHROWS = 32
_INV = np.float32(255.0 / (2.0 * np.pi))
_HALF = np.float32(0.5)
_LO = np.float32(0.0)
_HI = np.float32(255.0)


def _bucketize_body(x_hbm, lut_hbm, out_hbm, lut_v,
                    xb0, xb1, ob0, ob1, si0, si1, so0, so1):
    wid = lax.axis_index("s") * _NC + lax.axis_index("c")
    ncols = x_hbm.shape[1]
    rows_per_w = x_hbm.shape[0] // _NW
    row0 = wid * rows_per_w
    nch = rows_per_w // _CHROWS
    nvec = ncols // _L
    pltpu.sync_copy(lut_hbm, lut_v)

    xbs, obs, sis, sos = (xb0, xb1), (ob0, ob1), (si0, si1), (so0, so1)

    def in_slice(idx):
        return x_hbm.at[pl.ds(row0 + idx * _CHROWS, _CHROWS), :]

    def out_slice(idx):
        return out_hbm.at[pl.ds(row0 + idx * _CHROWS, _CHROWS), :]

    pltpu.async_copy(in_slice(0), xb0, si0)
    pltpu.async_copy(in_slice(1), xb1, si1)

    def pair(g, carry):
        for b in range(2):
            idx = g * 2 + b
            xb, ob, si, so = xbs[b], obs[b], sis[b], sos[b]
            pltpu.make_async_copy(in_slice(idx), xb, si).wait()

            @pl.when(idx >= 2)
            def _drain_out():
                pltpu.make_async_copy(ob, out_slice(idx - 2), so).wait()

            @plsc.parallel_loop(0, _CHROWS, step=1, unroll=2)
            def _compute(row):
                for j in range(nvec):
                    xv = xb[row, pl.ds(j * _L, _L)]
                    t = xv * _INV + _HALF
                    tc = jnp.minimum(jnp.maximum(t, _LO), _HI)
                    r = tc.astype(jnp.int32)
                    lv = plsc.load_gather(lut_v, [r])
                    ob[row, pl.ds(j * _L, _L)] = jnp.where(lv < xv, r + 1, r)

            pltpu.async_copy(ob, out_slice(idx), so)

            @pl.when(idx + 2 < nch)
            def _prefetch():
                pltpu.async_copy(in_slice(idx + 2), xb, si)

        return carry

    lax.fori_loop(0, nch // 2, pair, 0)
    pltpu.make_async_copy(ob0, out_slice(nch - 2), so0).wait()
    pltpu.make_async_copy(ob1, out_slice(nch - 1), so1).wait()


def kernel(x, phase_lut):
    nrows, ncols = x.shape
    assert nrows % (_NW * _CHROWS) == 0 and ncols % _L == 0
    assert phase_lut.shape == (256,)
    mesh = plsc.VectorSubcoreMesh(
        core_axis_name="c", subcore_axis_name="s",
        num_cores=_NC, num_subcores=_NS,
    )
    return pl.kernel(
        _bucketize_body,
        out_type=jax.ShapeDtypeStruct((nrows, ncols), jnp.int32),
        mesh=mesh,
        scratch_types=[
            pltpu.VMEM((256,), jnp.float32),
            pltpu.VMEM((_CHROWS, ncols), jnp.float32),
            pltpu.VMEM((_CHROWS, ncols), jnp.float32),
            pltpu.VMEM((_CHROWS, ncols), jnp.int32),
            pltpu.VMEM((_CHROWS, ncols), jnp.int32),
            pltpu.SemaphoreType.DMA,
            pltpu.SemaphoreType.DMA,
            pltpu.SemaphoreType.DMA,
            pltpu.SemaphoreType.DMA,
        ],
        compiler_params=pltpu.CompilerParams(
            needs_layout_passes=False,
        ),
    )(x, phase_lut)


# speedup vs baseline: 1.8660x; 1.8660x over previous
import jax
import jax.numpy as jnp
import numpy as np
from jax import lax
from jax.experimental import pallas as pl
from jax.experimental.pallas import tpu as pltpu
from jax.experimental.pallas import tpu_sc as plsc

_NC = 2
_NS = 16
_NW = _NC * _NS
_L = 16
_CHROWS = 32
_NVEC_SHIFT = 5
_INV = np.float32(255.0 / (2.0 * np.pi))
_HALF = np.float32(0.5)
_LO = np.float32(0.0)
_HI = np.float32(255.0)


def _bucketize_body(x_hbm, lut_hbm, out_hbm, lut_v,
                    xb0, xb1, ob0, ob1, si0, si1, so0, so1):
    wid = lax.axis_index("s") * _NC + lax.axis_index("c")
    ncols = x_hbm.shape[1]
    rows_per_w = x_hbm.shape[0] // _NW
    row0 = wid * rows_per_w
    nch = rows_per_w // _CHROWS
    nvec = ncols // _L
    assert nvec == 1 << _NVEC_SHIFT
    pltpu.sync_copy(lut_hbm, lut_v)

    xbs, obs, sis, sos = (xb0, xb1), (ob0, ob1), (si0, si1), (so0, so1)

    def in_slice(idx):
        return x_hbm.at[pl.ds(row0 + idx * _CHROWS, _CHROWS), :]

    def out_slice(idx):
        return out_hbm.at[pl.ds(row0 + idx * _CHROWS, _CHROWS), :]

    pltpu.async_copy(in_slice(0), xb0, si0)
    pltpu.async_copy(in_slice(1), xb1, si1)

    def pair(g, carry):
        for b in range(2):
            idx = g * 2 + b
            xb, ob, si, so = xbs[b], obs[b], sis[b], sos[b]
            pltpu.make_async_copy(in_slice(idx), xb, si).wait()

            @pl.when(idx >= 2)
            def _drain_out():
                pltpu.make_async_copy(ob, out_slice(idx - 2), so).wait()

            @plsc.parallel_loop(0, _CHROWS * nvec, step=1, unroll=8)
            def _compute(v):
                row = lax.shift_right_logical(v, _NVEC_SHIFT)
                col = lax.shift_left(jnp.bitwise_and(v, nvec - 1), 4)
                xv = xb[row, pl.ds(col, _L)]
                t = xv * _INV + _HALF
                tc = jnp.minimum(jnp.maximum(t, _LO), _HI)
                r = tc.astype(jnp.int32)
                lv = plsc.load_gather(lut_v, [r])
                ob[row, pl.ds(col, _L)] = jnp.where(lv < xv, r + 1, r)

            pltpu.async_copy(ob, out_slice(idx), so)

            @pl.when(idx + 2 < nch)
            def _prefetch():
                pltpu.async_copy(in_slice(idx + 2), xb, si)

        return carry

    lax.fori_loop(0, nch // 2, pair, 0)
    pltpu.make_async_copy(ob0, out_slice(nch - 2), so0).wait()
    pltpu.make_async_copy(ob1, out_slice(nch - 1), so1).wait()


def kernel(x, phase_lut):
    nrows, ncols = x.shape
    assert nrows % (_NW * _CHROWS) == 0 and ncols % _L == 0
    assert phase_lut.shape == (256,)
    mesh = plsc.VectorSubcoreMesh(
        core_axis_name="c", subcore_axis_name="s",
        num_cores=_NC, num_subcores=_NS,
    )
    return pl.kernel(
        _bucketize_body,
        out_type=jax.ShapeDtypeStruct((nrows, ncols), jnp.int32),
        mesh=mesh,
        scratch_types=[
            pltpu.VMEM((256,), jnp.float32),
            pltpu.VMEM((_CHROWS, ncols), jnp.float32),
            pltpu.VMEM((_CHROWS, ncols), jnp.float32),
            pltpu.VMEM((_CHROWS, ncols), jnp.int32),
            pltpu.VMEM((_CHROWS, ncols), jnp.int32),
            pltpu.SemaphoreType.DMA,
            pltpu.SemaphoreType.DMA,
            pltpu.SemaphoreType.DMA,
            pltpu.SemaphoreType.DMA,
        ],
        compiler_params=pltpu.CompilerParams(
            needs_layout_passes=False,
        ),
    )(x, phase_lut)
